# Initial kernel scaffold; baseline (speedup 1.0000x reference)
import jax
import jax.numpy as jnp
from jax.experimental import pallas as pl


def kernel(x, edge_index, params):
    raise NotImplementedError("write your pallas kernel here")


# speedup vs baseline: 1.6316x; 1.6316x over previous
import jax
import jax.numpy as jnp
from jax.experimental import pallas as pl


def _bn(x, g, b, m, v):
    return (x - m) / jnp.sqrt(v + 1e-5) * g + b


def _gat(x, src, dst, W, asrc, adst, b):
    n = x.shape[0]
    h = x @ W
    e = jax.nn.leaky_relu(jnp.sum(h * asrc, axis=-1)[src] + jnp.sum(h * adst, axis=-1)[dst], 0.2)
    ee = jnp.exp(e)
    num = jax.ops.segment_sum(h[src] * ee[:, None], dst, num_segments=n)
    denom = jax.ops.segment_sum(ee, dst, num_segments=n)
    return num / (denom[:, None] + 1e-16) + b


def _head_kernel(g_ref, w1_ref, b1_ref, bng_ref, bnb_ref, bnm_ref, bnv_ref,
                 w2_ref, b2_ref, out_ref):
    g = g_ref[...]
    h = jnp.maximum(jnp.dot(g, w1_ref[...], preferred_element_type=jnp.float32)
                    + b1_ref[...], 0.0)
    h = _bn(h, bng_ref[...], bnb_ref[...], bnm_ref[...], bnv_ref[...])
    out_ref[...] = jnp.dot(h, w2_ref[...], preferred_element_type=jnp.float32) + b2_ref[...]


def kernel(x, edge_index, params):
    p = params
    src = edge_index[0]
    dst = edge_index[1]
    x1 = jax.nn.relu(_gat(x, src, dst, p['conv1_W'], p['conv1_asrc'], p['conv1_adst'], p['conv1_b']))
    prev = _bn(x1, p['bn1_g'], p['bn1_b'], p['bn1_m'], p['bn1_v'])
    for i in range(2, 6):
        xi = jax.nn.relu(_gat(prev, src, dst, p['conv%d_W' % i], p['conv%d_asrc' % i],
                              p['conv%d_adst' % i], p['conv%d_b' % i]))
        xi = _bn(xi, p['bn%d_g' % i], p['bn%d_b' % i], p['bn%d_m' % i], p['bn%d_v' % i]) \
            + prev @ p['proj%d_W' % i] + p['proj%d_b' % i]
        prev = xi
    g = jnp.mean(prev, axis=0, keepdims=True)
    out = pl.pallas_call(
        _head_kernel,
        out_shape=jax.ShapeDtypeStruct((1, 1), jnp.float32),
    )(g, p['head_W1'], p['head_b1'][None, :],
      p['headbn_g'][None, :], p['headbn_b'][None, :],
      p['headbn_m'][None, :], p['headbn_v'][None, :],
      p['head_W2'], p['head_b2'][None, :])
    return out.reshape(-1)


# speedup vs baseline: 12.0063x; 7.3587x over previous
import functools

import jax
import jax.numpy as jnp
from jax import lax
from jax.experimental import pallas as pl
from jax.experimental.pallas import tpu as pltpu
from jax.experimental.pallas import tpu_sc as plsc

NN = 10000
NP = 10240
NE = 320000
D = 128
WSZ = 128
NWIN = NE // WSZ
NWORK = 32
HD = 64
ACC_W = 80


def _mm_body(prev, W, asrc, adst, h0, h1, ss, sd):
    hb = jnp.dot(prev[...], W[...], preferred_element_type=jnp.float32)
    h0[...] = hb[:, :HD]
    h1[...] = hb[:, HD:]
    ss[...] = jnp.sum(hb * asrc[...], axis=1)
    sd[...] = jnp.sum(hb * adst[...], axis=1)


_mm_call = pl.pallas_call(
    _mm_body,
    grid=(NP // 1024,),
    in_specs=[
        pl.BlockSpec((1024, D), lambda i: (i, 0)),
        pl.BlockSpec((D, D), lambda i: (0, 0)),
        pl.BlockSpec((1, D), lambda i: (0, 0)),
        pl.BlockSpec((1, D), lambda i: (0, 0)),
    ],
    out_specs=[
        pl.BlockSpec((1024, HD), lambda i: (i, 0)),
        pl.BlockSpec((1024, HD), lambda i: (i, 0)),
        pl.BlockSpec((1024,), lambda i: (i,)),
        pl.BlockSpec((1024,), lambda i: (i,)),
    ],
    out_shape=[
        jax.ShapeDtypeStruct((NP, HD), jnp.float32),
        jax.ShapeDtypeStruct((NP, HD), jnp.float32),
        jax.ShapeDtypeStruct((NP,), jnp.float32),
        jax.ShapeDtypeStruct((NP,), jnp.float32),
    ],
)


def _combine1_body(o0, o1, b, g, bb, m, v, out):
    U = jnp.concatenate([o0[:NN, :HD], o1[:NN, :HD]], axis=1)
    Dn = o0[:NN, HD:HD + 1] + 0.0 * o1[:NN, HD:HD + 1]
    xi = jnp.maximum(U / (Dn + 1e-16) + b[...], 0.0)
    xi = (xi - m[...]) / jnp.sqrt(v[...] + 1e-5) * g[...] + bb[...]
    out[pl.ds(0, NN), :] = xi
    out[pl.ds(NN, NP - NN), :] = jnp.zeros((NP - NN, D), jnp.float32)


_combine1_call = pl.pallas_call(
    _combine1_body,
    out_shape=jax.ShapeDtypeStruct((NP, D), jnp.float32),
)


def _combine_body(o0, o1, b, g, bb, m, v, prev, pW, pb, out):
    U = jnp.concatenate([o0[:NN, :HD], o1[:NN, :HD]], axis=1)
    Dn = o0[:NN, HD:HD + 1] + 0.0 * o1[:NN, HD:HD + 1]
    xi = jnp.maximum(U / (Dn + 1e-16) + b[...], 0.0)
    xi = (xi - m[...]) / jnp.sqrt(v[...] + 1e-5) * g[...] + bb[...]
    res = jnp.dot(prev[...], pW[...], preferred_element_type=jnp.float32)
    out[pl.ds(0, NN), :] = xi + res[:NN, :] + pb[...]
    out[pl.ds(NN, NP - NN), :] = jnp.zeros((NP - NN, D), jnp.float32)


_combine_call = pl.pallas_call(
    _combine_body,
    out_shape=jax.ShapeDtypeStruct((NP, D), jnp.float32),
)


def _head_body(prev, w1, b1, g, bb, m, v, w2, b2, out):
    x = prev[...]
    mask = lax.broadcasted_iota(jnp.int32, (NP, 1), 0) < NN
    x = jnp.where(mask, x, 0.0)
    gmean = (jnp.sum(x, axis=0, keepdims=True) / NN)
    h = jnp.maximum(jnp.dot(gmean, w1[...], preferred_element_type=jnp.float32) + b1[...], 0.0)
    h = (h - m[...]) / jnp.sqrt(v[...] + 1e-5) * g[...] + bb[...]
    out[...] = jnp.dot(h, w2[...], preferred_element_type=jnp.float32) + b2[...]


_head_call = pl.pallas_call(
    _head_body,
    out_shape=jax.ShapeDtypeStruct((1, 1), jnp.float32),
)


def _edge_body(h0_hbm, h1_hbm, ssrc_hbm, sdst_hbm, src_hbm, dst_hbm, out0, out1,
               ssrc_v, sdst_v, srcw, dstw, rows, upd, eew, acc):
    c = lax.axis_index("c")
    s = lax.axis_index("s")

    pltpu.sync_copy(ssrc_hbm, ssrc_v)
    pltpu.sync_copy(sdst_hbm, sdst_v)

    zero = jnp.zeros((16,), jnp.float32)

    def zrow(i, _):
        for j in range(ACC_W // 16):
            upd[i, pl.ds(j * 16, 16)] = zero
        return 0

    lax.fori_loop(0, WSZ, zrow, 0)

    zb = s * (NP // 16)
    for k in range(5):
        pltpu.sync_copy(upd, acc.at[pl.ds(zb + k * 128, 128), :])
    plsc.subcore_barrier()

    npc = NWIN // 16
    n_i = npc + jnp.where(s < NWIN - npc * 16, 1, 0)

    def window(i, _):
        base = (s + i * 16) * WSZ
        pltpu.sync_copy(src_hbm.at[pl.ds(base, WSZ)], srcw)
        pltpu.sync_copy(dst_hbm.at[pl.ds(base, WSZ)], dstw)

        @pl.when(c == 0)
        def _():
            pltpu.sync_copy(h0_hbm.at[srcw], rows)

        @pl.when(c == 1)
        def _():
            pltpu.sync_copy(h1_hbm.at[srcw], rows)
        for g in range(WSZ // 16):
            si = srcw[pl.ds(g * 16, 16)]
            di = dstw[pl.ds(g * 16, 16)]
            s1 = plsc.load_gather(ssrc_v, [si])
            s2 = plsc.load_gather(sdst_v, [di])
            e = s1 + s2
            e = jnp.maximum(e, 0.2 * e)
            eew[pl.ds(g * 16, 16)] = jnp.exp(e)

        lane0 = lax.iota(jnp.int32, 16) == 0

        def edge(el, _):
            eb = plsc.load_gather(eew, [jnp.full((16,), el, jnp.int32)])
            for j in range(HD // 16):
                upd[el, pl.ds(j * 16, 16)] = eb * rows[el, pl.ds(j * 16, 16)]
            upd[el, pl.ds(HD, 16)] = jnp.where(lane0, eb, 0.0)
            return 0

        lax.fori_loop(0, WSZ, edge, 0)
        pltpu.sync_copy(upd, acc.at[dstw], add=True)
        return 0

    lax.fori_loop(0, n_i, window, 0)
    plsc.subcore_barrier()

    for k in range(5):
        sl = pl.ds(zb + k * 128, 128)

        @pl.when(c == 0)
        def _():
            pltpu.sync_copy(acc.at[sl, :], out0.at[sl, :])

        @pl.when(c == 1)
        def _():
            pltpu.sync_copy(acc.at[sl, :], out1.at[sl, :])


_edge_call = pl.kernel(
    _edge_body,
    out_type=(
        jax.ShapeDtypeStruct((NP, ACC_W), jnp.float32),
        jax.ShapeDtypeStruct((NP, ACC_W), jnp.float32),
    ),
    mesh=plsc.VectorSubcoreMesh(core_axis_name="c", subcore_axis_name="s",
                                num_cores=2, num_subcores=16),
    compiler_params=pltpu.CompilerParams(needs_layout_passes=False,
                                         use_tc_tiling_on_sc=False),
    scratch_types=[
        pltpu.VMEM((NP,), jnp.float32),
        pltpu.VMEM((NP,), jnp.float32),
        pltpu.VMEM((WSZ,), jnp.int32),
        pltpu.VMEM((WSZ,), jnp.int32),
        pltpu.VMEM((WSZ, HD), jnp.float32),
        pltpu.VMEM((WSZ, ACC_W), jnp.float32),
        pltpu.VMEM((WSZ,), jnp.float32),
        pltpu.VMEM_SHARED((NP, ACC_W), jnp.float32),
    ],
)


def kernel(x, edge_index, params):
    p = params
    src = edge_index[0]
    dst = edge_index[1]
    xp = jnp.zeros((NP, D), jnp.float32).at[:NN].set(x)

    r2 = lambda a: a.reshape(1, D)
    prev = None
    for i in range(1, 6):
        W = p['conv%d_W' % i]
        h0, h1, ss, sd = _mm_call(xp if i == 1 else prev, W,
                                  r2(p['conv%d_asrc' % i]), r2(p['conv%d_adst' % i]))
        o0, o1 = _edge_call(h0, h1, ss, sd, src, dst)
        bn = (r2(p['bn%d_g' % i]), r2(p['bn%d_b' % i]),
              r2(p['bn%d_m' % i]), r2(p['bn%d_v' % i]))
        if i == 1:
            prev = _combine1_call(o0, o1, r2(p['conv1_b']), *bn)
        else:
            prev = _combine_call(o0, o1, r2(p['conv%d_b' % i]), *bn,
                                 prev, p['proj%d_W' % i], r2(p['proj%d_b' % i]))

    out = _head_call(prev, p['head_W1'], p['head_b1'][None, :],
                     p['headbn_g'][None, :], p['headbn_b'][None, :],
                     p['headbn_m'][None, :], p['headbn_v'][None, :],
                     p['head_W2'], p['head_b2'][None, :])
    return out.reshape(-1)


# speedup vs baseline: 28.7780x; 2.3969x over previous
import functools

import jax
import jax.numpy as jnp
from jax import lax
from jax.experimental import pallas as pl
from jax.experimental.pallas import tpu as pltpu
from jax.experimental.pallas import tpu_sc as plsc

NN = 10000
NP = 10240
NE = 320000
D = 128
WSZ = 128
NWT = 158
NEP = 16 * NWT * WSZ
HD = 64
ACC_W = 80


def _mm_body(prev, W, asrc, adst, h0, h1, ss, sd):
    hb = jnp.dot(prev[...], W[...], preferred_element_type=jnp.float32)
    h0[...] = hb[:, :HD]
    h1[...] = hb[:, HD:]
    ss[...] = jnp.sum(hb * asrc[...], axis=1)
    sd[...] = jnp.sum(hb * adst[...], axis=1)


_mm_call = pl.pallas_call(
    _mm_body,
    grid=(NP // 1024,),
    in_specs=[
        pl.BlockSpec((1024, D), lambda i: (i, 0)),
        pl.BlockSpec((D, D), lambda i: (0, 0)),
        pl.BlockSpec((1, D), lambda i: (0, 0)),
        pl.BlockSpec((1, D), lambda i: (0, 0)),
    ],
    out_specs=[
        pl.BlockSpec((1024, HD), lambda i: (i, 0)),
        pl.BlockSpec((1024, HD), lambda i: (i, 0)),
        pl.BlockSpec((1024,), lambda i: (i,)),
        pl.BlockSpec((1024,), lambda i: (i,)),
    ],
    out_shape=[
        jax.ShapeDtypeStruct((NP, HD), jnp.float32),
        jax.ShapeDtypeStruct((NP, HD), jnp.float32),
        jax.ShapeDtypeStruct((NP,), jnp.float32),
        jax.ShapeDtypeStruct((NP,), jnp.float32),
    ],
)


def _combine1_body(o0, o1, b, g, bb, m, v, out):
    U = jnp.concatenate([o0[:NN, :HD], o1[:NN, :HD]], axis=1)
    Dn = o0[:NN, HD:HD + 1] + 0.0 * o1[:NN, HD:HD + 1]
    xi = jnp.maximum(U / (Dn + 1e-16) + b[...], 0.0)
    xi = (xi - m[...]) / jnp.sqrt(v[...] + 1e-5) * g[...] + bb[...]
    out[pl.ds(0, NN), :] = xi
    out[pl.ds(NN, NP - NN), :] = jnp.zeros((NP - NN, D), jnp.float32)


_combine1_call = pl.pallas_call(
    _combine1_body,
    out_shape=jax.ShapeDtypeStruct((NP, D), jnp.float32),
)


def _combine_body(o0, o1, b, g, bb, m, v, prev, pW, pb, out):
    U = jnp.concatenate([o0[:NN, :HD], o1[:NN, :HD]], axis=1)
    Dn = o0[:NN, HD:HD + 1] + 0.0 * o1[:NN, HD:HD + 1]
    xi = jnp.maximum(U / (Dn + 1e-16) + b[...], 0.0)
    xi = (xi - m[...]) / jnp.sqrt(v[...] + 1e-5) * g[...] + bb[...]
    res = jnp.dot(prev[...], pW[...], preferred_element_type=jnp.float32)
    out[pl.ds(0, NN), :] = xi + res[:NN, :] + pb[...]
    out[pl.ds(NN, NP - NN), :] = jnp.zeros((NP - NN, D), jnp.float32)


_combine_call = pl.pallas_call(
    _combine_body,
    out_shape=jax.ShapeDtypeStruct((NP, D), jnp.float32),
)


def _head_body(prev, w1, b1, g, bb, m, v, w2, b2, out):
    x = prev[...]
    mask = lax.broadcasted_iota(jnp.int32, (NP, 1), 0) < NN
    x = jnp.where(mask, x, 0.0)
    gmean = (jnp.sum(x, axis=0, keepdims=True) / NN)
    h = jnp.maximum(jnp.dot(gmean, w1[...], preferred_element_type=jnp.float32) + b1[...], 0.0)
    h = (h - m[...]) / jnp.sqrt(v[...] + 1e-5) * g[...] + bb[...]
    out[...] = jnp.dot(h, w2[...], preferred_element_type=jnp.float32) + b2[...]


_head_call = pl.pallas_call(
    _head_body,
    out_shape=jax.ShapeDtypeStruct((1, 1), jnp.float32),
)


def _edge_body(h0_hbm, h1_hbm, ssrc_hbm, sdst_hbm, src_hbm, dst_hbm, out0, out1,
               ssrc_v, sdst_v, srcA, srcB, dstA, dstB, rowsA, rowsB,
               updA, updB, eew, acc, semi, semgA, semgB):
    c = lax.axis_index("c")
    s = lax.axis_index("s")

    pltpu.sync_copy(ssrc_hbm, ssrc_v)
    pltpu.sync_copy(sdst_hbm, sdst_v)

    zero = jnp.zeros((16,), jnp.float32)

    @plsc.parallel_loop(0, WSZ, unroll=4)
    def _(i):
        for j in range(ACC_W // 16):
            updA[i, pl.ds(j * 16, 16)] = zero

    zb = s * (NP // 16)
    for k in range(5):
        pltpu.sync_copy(updA, acc.at[pl.ds(zb + k * 128, 128), :])
    plsc.subcore_barrier()

    lane0 = lax.iota(jnp.int32, 16) == 0

    def process(srcX, dstX, rowsX, updX):
        for g in range(WSZ // 16):
            si = srcX[pl.ds(g * 16, 16)]
            di = dstX[pl.ds(g * 16, 16)]
            e = plsc.load_gather(ssrc_v, [si]) + plsc.load_gather(sdst_v, [di])
            e = jnp.maximum(e, 0.2 * e)
            eew[pl.ds(g * 16, 16)] = jnp.exp(e)

        @plsc.parallel_loop(0, WSZ, unroll=4)
        def _(el):
            eb = plsc.load_gather(eew, [jnp.full((16,), el, jnp.int32)])
            for j in range(HD // 16):
                updX[el, pl.ds(j * 16, 16)] = eb * rowsX[el, pl.ds(j * 16, 16)]
            updX[el, pl.ds(HD, 16)] = jnp.where(lane0, eb, 0.0)

        pltpu.sync_copy(updX, acc.at[dstX], add=True)

    tb = s * NWT

    def pair(i, _):
        b0 = (tb + 2 * i) * WSZ
        d1 = pltpu.async_copy(src_hbm.at[pl.ds(b0, WSZ)], srcA, semi)
        d2 = pltpu.async_copy(dst_hbm.at[pl.ds(b0, WSZ)], dstA, semi)
        d3 = pltpu.async_copy(src_hbm.at[pl.ds(b0 + WSZ, WSZ)], srcB, semi)
        d4 = pltpu.async_copy(dst_hbm.at[pl.ds(b0 + WSZ, WSZ)], dstB, semi)
        d1.wait()
        d2.wait()
        d3.wait()
        d4.wait()

        @pl.when(c == 0)
        def _():
            pltpu.async_copy(h0_hbm.at[srcA], rowsA, semgA)
            pltpu.async_copy(h0_hbm.at[srcB], rowsB, semgB)

        @pl.when(c == 1)
        def _():
            pltpu.async_copy(h1_hbm.at[srcA], rowsA, semgA)
            pltpu.async_copy(h1_hbm.at[srcB], rowsB, semgB)
        pltpu.make_async_copy(h0_hbm.at[srcA], rowsA, semgA).wait()
        process(srcA, dstA, rowsA, updA)
        pltpu.make_async_copy(h0_hbm.at[srcB], rowsB, semgB).wait()
        process(srcB, dstB, rowsB, updB)
        return 0

    lax.fori_loop(0, NWT // 2, pair, 0)
    plsc.subcore_barrier()

    for k in range(5):
        sl = pl.ds(zb + k * 128, 128)

        @pl.when(c == 0)
        def _():
            pltpu.sync_copy(acc.at[sl, :], out0.at[sl, :])

        @pl.when(c == 1)
        def _():
            pltpu.sync_copy(acc.at[sl, :], out1.at[sl, :])


_edge_call = pl.kernel(
    _edge_body,
    out_type=(
        jax.ShapeDtypeStruct((NP, ACC_W), jnp.float32),
        jax.ShapeDtypeStruct((NP, ACC_W), jnp.float32),
    ),
    mesh=plsc.VectorSubcoreMesh(core_axis_name="c", subcore_axis_name="s",
                                num_cores=2, num_subcores=16),
    compiler_params=pltpu.CompilerParams(needs_layout_passes=False,
                                         use_tc_tiling_on_sc=False),
    scratch_types=[
        pltpu.VMEM((NP,), jnp.float32),
        pltpu.VMEM((NP,), jnp.float32),
        pltpu.VMEM((WSZ,), jnp.int32),
        pltpu.VMEM((WSZ,), jnp.int32),
        pltpu.VMEM((WSZ,), jnp.int32),
        pltpu.VMEM((WSZ,), jnp.int32),
        pltpu.VMEM((WSZ, HD), jnp.float32),
        pltpu.VMEM((WSZ, HD), jnp.float32),
        pltpu.VMEM((WSZ, ACC_W), jnp.float32),
        pltpu.VMEM((WSZ, ACC_W), jnp.float32),
        pltpu.VMEM((WSZ,), jnp.float32),
        pltpu.VMEM_SHARED((NP, ACC_W), jnp.float32),
        pltpu.SemaphoreType.DMA,
        pltpu.SemaphoreType.DMA,
        pltpu.SemaphoreType.DMA,
    ],
)


def kernel(x, edge_index, params):
    p = params
    pade = NEP - NE
    pidx = jnp.arange(pade, dtype=jnp.int32)
    src = jnp.concatenate([edge_index[0], (pidx * 97) % NN])
    dst = jnp.concatenate([edge_index[1], NN + (pidx % (NP - NN))])
    xp = jnp.zeros((NP, D), jnp.float32).at[:NN].set(x)

    r2 = lambda a: a.reshape(1, D)
    prev = None
    for i in range(1, 6):
        W = p['conv%d_W' % i]
        h0, h1, ss, sd = _mm_call(xp if i == 1 else prev, W,
                                  r2(p['conv%d_asrc' % i]), r2(p['conv%d_adst' % i]))
        o0, o1 = _edge_call(h0, h1, ss, sd, src, dst)
        bn = (r2(p['bn%d_g' % i]), r2(p['bn%d_b' % i]),
              r2(p['bn%d_m' % i]), r2(p['bn%d_v' % i]))
        if i == 1:
            prev = _combine1_call(o0, o1, r2(p['conv1_b']), *bn)
        else:
            prev = _combine_call(o0, o1, r2(p['conv%d_b' % i]), *bn,
                                 prev, p['proj%d_W' % i], r2(p['proj%d_b' % i]))

    out = _head_call(prev, p['head_W1'], p['head_b1'][None, :],
                     p['headbn_g'][None, :], p['headbn_b'][None, :],
                     p['headbn_m'][None, :], p['headbn_v'][None, :],
                     p['head_W2'], p['head_b2'][None, :])
    return out.reshape(-1)


# speedup vs baseline: 46.1792x; 1.6047x over previous
import jax
import jax.numpy as jnp
from jax import lax
from jax.experimental import pallas as pl
from jax.experimental.pallas import tpu as pltpu
from jax.experimental.pallas import tpu_sc as plsc

NN = 10000
NP = 10240
NE = 320000
D = 128
WSZ = 128
NWT = 159
NQ = NWT // 3
NEP = 16 * NWT * WSZ
HD = 64
ACC_W = 80


def _mm_body(prev, W, asrc, adst, h0, h1, ss, sd):
    hb = jnp.dot(prev[...], W[...], preferred_element_type=jnp.float32)
    h0[...] = hb[:, :HD]
    h1[...] = hb[:, HD:]
    ss[...] = jnp.sum(hb * asrc[...], axis=1)
    sd[...] = jnp.sum(hb * adst[...], axis=1)


_mm_call = pl.pallas_call(
    _mm_body,
    grid=(NP // 1024,),
    in_specs=[
        pl.BlockSpec((1024, D), lambda i: (i, 0)),
        pl.BlockSpec((D, D), lambda i: (0, 0)),
        pl.BlockSpec((1, D), lambda i: (0, 0)),
        pl.BlockSpec((1, D), lambda i: (0, 0)),
    ],
    out_specs=[
        pl.BlockSpec((1024, HD), lambda i: (i, 0)),
        pl.BlockSpec((1024, HD), lambda i: (i, 0)),
        pl.BlockSpec((1024,), lambda i: (i,)),
        pl.BlockSpec((1024,), lambda i: (i,)),
    ],
    out_shape=[
        jax.ShapeDtypeStruct((NP, HD), jnp.float32),
        jax.ShapeDtypeStruct((NP, HD), jnp.float32),
        jax.ShapeDtypeStruct((NP,), jnp.float32),
        jax.ShapeDtypeStruct((NP,), jnp.float32),
    ],
)


def _norm_bn(o0, o1, b, g, bb, m, v):
    U = jnp.concatenate([o0[:NN, :HD], o1[:NN, :HD]], axis=1)
    Dn = o0[:NN, HD:HD + 1]
    xi = jnp.maximum(U / (Dn + 1e-16) + b[...], 0.0)
    return (xi - m[...]) / jnp.sqrt(v[...] + 1e-5) * g[...] + bb[...]


def _fuse_body(o0, o1, b, g, bb, m, v, prev, pW, pb, W2, a2s, a2d,
               prevnew, h0, h1, ss, sd):
    xi = _norm_bn(o0, o1, b, g, bb, m, v)
    res = jnp.dot(prev[...], pW[...], preferred_element_type=jnp.float32)
    pn = jnp.concatenate(
        [xi + res[:NN, :] + pb[...], jnp.zeros((NP - NN, D), jnp.float32)], axis=0)
    prevnew[...] = pn
    hb = jnp.dot(pn, W2[...], preferred_element_type=jnp.float32)
    h0[...] = hb[:, :HD]
    h1[...] = hb[:, HD:]
    ss[...] = jnp.sum(hb * a2s[...], axis=1)
    sd[...] = jnp.sum(hb * a2d[...], axis=1)


_fuse_call = pl.pallas_call(
    _fuse_body,
    out_shape=[
        jax.ShapeDtypeStruct((NP, D), jnp.float32),
        jax.ShapeDtypeStruct((NP, HD), jnp.float32),
        jax.ShapeDtypeStruct((NP, HD), jnp.float32),
        jax.ShapeDtypeStruct((NP,), jnp.float32),
        jax.ShapeDtypeStruct((NP,), jnp.float32),
    ],
)


def _fuse5_body(o0, o1, b, g, bb, m, v, prev, pW, pb,
                w1, b1, hg, hbb, hm, hv, w2, b2, out):
    xi = _norm_bn(o0, o1, b, g, bb, m, v)
    res = jnp.dot(prev[...], pW[...], preferred_element_type=jnp.float32)
    pn = xi + res[:NN, :] + pb[...]
    gmean = jnp.sum(pn, axis=0, keepdims=True) / NN
    h = jnp.maximum(jnp.dot(gmean, w1[...], preferred_element_type=jnp.float32)
                    + b1[...], 0.0)
    h = (h - hm[...]) / jnp.sqrt(hv[...] + 1e-5) * hg[...] + hbb[...]
    out[...] = jnp.dot(h, w2[...], preferred_element_type=jnp.float32) + b2[...]


_fuse5_call = pl.pallas_call(
    _fuse5_body,
    out_shape=jax.ShapeDtypeStruct((1, 1), jnp.float32),
)


def _edge_body(h0_hbm, h1_hbm, ssrc_hbm, sdst_hbm, src_hbm, dst_hbm, out0, out1,
               ssrc_v, sdst_v,
               src0, src1, src2, dst0, dst1, dst2,
               rows0, rows1, rows2, upd0, upd1, upd2, eew, acc,
               semi0, semi1, semi2, semg0, semg1, semg2,
               sems0, sems1, sems2):
    c = lax.axis_index("c")
    s = lax.axis_index("s")
    SRC = [src0, src1, src2]
    DST = [dst0, dst1, dst2]
    ROWS = [rows0, rows1, rows2]
    UPD = [upd0, upd1, upd2]
    SEMI = [semi0, semi1, semi2]
    SEMG = [semg0, semg1, semg2]
    SEMS = [sems0, sems1, sems2]

    pltpu.sync_copy(ssrc_hbm, ssrc_v)
    pltpu.sync_copy(sdst_hbm, sdst_v)

    zero = jnp.zeros((16,), jnp.float32)

    @plsc.parallel_loop(0, WSZ, unroll=4)
    def _(i):
        for j in range(ACC_W // 16):
            upd0[i, pl.ds(j * 16, 16)] = zero

    zb = s * (NP // 16)
    for k in range(5):
        pltpu.sync_copy(upd0, acc.at[pl.ds(zb + k * 128, 128), :])
    plsc.subcore_barrier()

    lane0 = lax.iota(jnp.int32, 16) == 0
    tb = s * NWT

    def fetch_idx(w, j):
        b = (tb + w) * WSZ
        pltpu.async_copy(src_hbm.at[pl.ds(b, WSZ)], SRC[j], SEMI[j])
        pltpu.async_copy(dst_hbm.at[pl.ds(b, WSZ)], DST[j], SEMI[j])

    def wait_idx(j):
        pltpu.make_async_copy(src_hbm.at[pl.ds(0, WSZ)], SRC[j], SEMI[j]).wait()
        pltpu.make_async_copy(dst_hbm.at[pl.ds(0, WSZ)], DST[j], SEMI[j]).wait()

    def issue_gather(j):
        @pl.when(c == 0)
        def _():
            pltpu.async_copy(h0_hbm.at[SRC[j]], ROWS[j], SEMG[j])

        @pl.when(c == 1)
        def _():
            pltpu.async_copy(h1_hbm.at[SRC[j]], ROWS[j], SEMG[j])

    def wait_gather(j):
        pltpu.make_async_copy(h0_hbm.at[SRC[j]], ROWS[j], SEMG[j]).wait()

    def wait_scatter(j):
        pltpu.make_async_copy(UPD[j], acc.at[DST[j]], SEMS[j]).wait()

    def compute(j):
        srcX, dstX, rowsX, updX = SRC[j], DST[j], ROWS[j], UPD[j]
        for g in range(WSZ // 16):
            si = srcX[pl.ds(g * 16, 16)]
            di = dstX[pl.ds(g * 16, 16)]
            e = plsc.load_gather(ssrc_v, [si]) + plsc.load_gather(sdst_v, [di])
            e = jnp.maximum(e, 0.2 * e)
            eew[pl.ds(g * 16, 16)] = jnp.exp(e)

        @plsc.parallel_loop(0, WSZ, unroll=4)
        def _(el):
            eb = plsc.load_gather(eew, [jnp.full((16,), el, jnp.int32)])
            for j2 in range(HD // 16):
                updX[el, pl.ds(j2 * 16, 16)] = eb * rowsX[el, pl.ds(j2 * 16, 16)]
            updX[el, pl.ds(HD, 16)] = jnp.where(lane0, eb, 0.0)

        pltpu.async_copy(updX, acc.at[dstX], SEMS[j], add=True)

    fetch_idx(0, 0)
    fetch_idx(1, 1)
    wait_idx(0)
    issue_gather(0)

    def triple(q, _):
        for k in range(3):
            jp = (k - 1) % 3
            jg = (k - 2) % 3

            if k == 0:
                @pl.when(q > 0)
                def _():
                    wait_scatter(jp)
            else:
                wait_scatter(jp)
            wf = 3 * q + k + 2
            if k == 0:
                fetch_idx(wf, jp)
            else:
                @pl.when(wf < NWT)
                def _():
                    fetch_idx(wf, jp)

            wg = 3 * q + k + 1
            if k == 2:
                @pl.when(wg < NWT)
                def _():
                    wait_idx(jg)
                    issue_gather(jg)
            else:
                wait_idx(jg)
                issue_gather(jg)

            wait_gather(k)
            compute(k)
        return 0

    lax.fori_loop(0, NQ, triple, 0)
    wait_scatter(2)
    plsc.subcore_barrier()

    for k in range(5):
        sl = pl.ds(zb + k * 128, 128)

        @pl.when(c == 0)
        def _():
            pltpu.sync_copy(acc.at[sl, :], out0.at[sl, :])

        @pl.when(c == 1)
        def _():
            pltpu.sync_copy(acc.at[sl, :], out1.at[sl, :])


_edge_call = pl.kernel(
    _edge_body,
    out_type=(
        jax.ShapeDtypeStruct((NP, ACC_W), jnp.float32),
        jax.ShapeDtypeStruct((NP, ACC_W), jnp.float32),
    ),
    mesh=plsc.VectorSubcoreMesh(core_axis_name="c", subcore_axis_name="s",
                                num_cores=2, num_subcores=16),
    compiler_params=pltpu.CompilerParams(needs_layout_passes=False,
                                         use_tc_tiling_on_sc=False),
    scratch_types=(
        [pltpu.VMEM((NP,), jnp.float32)] * 2
        + [pltpu.VMEM((WSZ,), jnp.int32)] * 6
        + [pltpu.VMEM((WSZ, HD), jnp.float32)] * 3
        + [pltpu.VMEM((WSZ, ACC_W), jnp.float32)] * 3
        + [pltpu.VMEM((WSZ,), jnp.float32)]
        + [pltpu.VMEM_SHARED((NP, ACC_W), jnp.float32)]
        + [pltpu.SemaphoreType.DMA] * 9
    ),
)


def kernel(x, edge_index, params):
    p = params
    pade = NEP - NE
    pidx = jnp.arange(pade, dtype=jnp.int32)
    src = jnp.concatenate([edge_index[0], (pidx * 97) % NN])
    dst = jnp.concatenate([edge_index[1], NN + (pidx % (NP - NN))])
    xp = jnp.zeros((NP, D), jnp.float32).at[:NN].set(x)

    r2 = lambda a: a.reshape(1, D)
    zW = jnp.zeros((D, D), jnp.float32)
    zb = jnp.zeros((1, D), jnp.float32)

    h0, h1, ss, sd = _mm_call(xp, p['conv1_W'], r2(p['conv1_asrc']), r2(p['conv1_adst']))
    prev = xp
    for i in range(1, 5):
        o0, o1 = _edge_call(h0, h1, ss, sd, src, dst)
        bn = (r2(p['bn%d_g' % i]), r2(p['bn%d_b' % i]),
              r2(p['bn%d_m' % i]), r2(p['bn%d_v' % i]))
        pW = zW if i == 1 else p['proj%d_W' % i]
        pb = zb if i == 1 else r2(p['proj%d_b' % i])
        j = i + 1
        prev, h0, h1, ss, sd = _fuse_call(
            o0, o1, r2(p['conv%d_b' % i]), *bn, prev, pW, pb,
            p['conv%d_W' % j], r2(p['conv%d_asrc' % j]), r2(p['conv%d_adst' % j]))

    o0, o1 = _edge_call(h0, h1, ss, sd, src, dst)
    bn5 = (r2(p['bn5_g']), r2(p['bn5_b']), r2(p['bn5_m']), r2(p['bn5_v']))
    out = _fuse5_call(o0, o1, r2(p['conv5_b']), *bn5, prev,
                      p['proj5_W'], r2(p['proj5_b']),
                      p['head_W1'], p['head_b1'][None, :],
                      p['headbn_g'][None, :], p['headbn_b'][None, :],
                      p['headbn_m'][None, :], p['headbn_v'][None, :],
                      p['head_W2'], p['head_b2'][None, :])
    return out.reshape(-1)
